# baseline (speedup 1.0000x reference)
import numpy as np
import jax
import jax.numpy as jnp
from jax.experimental import pallas as pl
from jax.experimental.pallas import tpu as pltpu

_B = 4096
_D_IN = 1600
_D_LAT = 64
_NC = 9
_NCP = 16
_BLK = 512
_NBLK = _B // _BLK

_C00 = (((0,), (0,)), ((), ()))
_STD = (((1,), (0,)), ((), ()))


def _neighbor_selection_mats_t():
    specs = [
        lambda k1, k2: (k1, k2),
        lambda k1, k2: (k1 + 1, k2) if k1 < 2 else None,
        lambda k1, k2: (k1 - 1, k2) if k1 > 0 else None,
        lambda k1, k2: (k1, k2 + 1) if k2 < 2 else None,
        lambda k1, k2: (k1, k2 - 1) if k2 > 0 else None,
    ]
    out = np.zeros((_NCP, 5 * _NCP), np.float32)
    for s, f in enumerate(specs):
        for j in range(_NC):
            r = f(j // 3, j % 3)
            if r is not None:
                out[r[0] * 3 + r[1], s * _NCP + j] = 1.0
    return out


_NBMT = _neighbor_selection_mats_t()


def _dot(a, b, dims):
    return jax.lax.dot_general(a, b, dims, preferred_element_type=jnp.float32)


def _fwd_kernel(xt_ref, eW1, eb1, eW2, eb2, eW3t, eb3,
                dW1, db1, dW2, db2, dW3t, db3,
                qW1, qb1, qW2, qb2, qW3t, qb3,
                embt_ref, nbmt_ref,
                loss_ref, recon_ref,
                xq_tab, nb_tab, acc):
    pid = pl.program_id(0)

    @pl.when(pid == 0)
    def _init():
        et = embt_ref[:]
        hq = jnp.maximum(_dot(qW1[:], et, _C00) + qb1[:], 0.0)
        hq = jnp.maximum(_dot(qW2[:], hq, _C00) + qb2[:], 0.0)
        xq_tab[:] = _dot(qW3t[:], hq, _STD) + qb3[:]
        nb_tab[:] = jnp.concatenate(
            [_dot(et, nbmt_ref[:, s * _NCP:(s + 1) * _NCP], _STD)
             for s in range(5)], axis=0)
        acc[0] = 0.0
        acc[1] = 0.0
        acc[2] = 0.0
        acc[3] = 0.0

    xt = xt_ref[:]
    h = jnp.maximum(_dot(eW1[:], xt, _C00) + eb1[:], 0.0)
    h = jnp.maximum(_dot(eW2[:], h, _C00) + eb2[:], 0.0)
    zt = jnp.tanh(_dot(eW3t[:], h, _STD) + eb3[:])

    et = embt_ref[:]
    best_d = jnp.sum((zt - et[:, 0:1]) ** 2, axis=0, keepdims=True)
    best_k = jnp.zeros((1, _BLK), jnp.int32)
    for j in range(1, _NC):
        dj = jnp.sum((zt - et[:, j:j + 1]) ** 2, axis=0, keepdims=True)
        upd = dj < best_d
        best_d = jnp.where(upd, dj, best_d)
        best_k = jnp.where(upd, j, best_k)

    onehot = (best_k == jax.lax.broadcasted_iota(
        jnp.int32, (_NCP, _BLK), 0)).astype(jnp.float32)

    xq = _dot(xq_tab[:], onehot, _STD)
    nb = _dot(nb_tab[:], onehot, _STD)
    zq = nb[0:_D_LAT, :]

    hd = jnp.maximum(
        _dot(dW1[:], zt.astype(jnp.bfloat16), _C00) + db1[:], 0.0)
    hd = jnp.maximum(
        _dot(dW2[:], hd.astype(jnp.bfloat16), _C00) + db2[:], 0.0)
    xe = _dot(dW3t[:], hd.astype(jnp.bfloat16), _STD) + db3[:]

    recon_ref[:] = xq + xe

    z5 = jnp.concatenate([zt] * 5, axis=0)
    acc[0] += jnp.sum((xt - xe) ** 2)
    acc[1] += jnp.sum((xt - xq) ** 2)
    acc[2] += jnp.sum((zt - zq) ** 2)
    acc[3] += jnp.sum((z5 - nb) ** 2)

    @pl.when(pid == _NBLK - 1)
    def _fin():
        loss_val = ((acc[0] + acc[1]) / (_B * _D_IN)
                    + acc[2] / (_B * _D_LAT)
                    + acc[3] / (_B * 5 * _D_LAT))
        loss_ref[:, :] = jnp.full((1, 1), loss_val, jnp.float32)


def _const_spec(shape):
    return pl.BlockSpec(shape, lambda i: (0,) * len(shape))


def _forward_pallas(xt, eW1, eb1, eW2, eb2, eW3t, eb3,
                    dW1, db1, dW2, db2, dW3t, db3,
                    qW1, qb1, qW2, qb2, qW3t, qb3, embt, nbmt):
    in_specs = [pl.BlockSpec((_D_IN, _BLK), lambda i: (0, i))]
    for a in (eW1, eb1, eW2, eb2, eW3t, eb3, dW1, db1, dW2, db2, dW3t, db3,
              qW1, qb1, qW2, qb2, qW3t, qb3, embt, nbmt):
        in_specs.append(_const_spec(a.shape))
    loss, recon = pl.pallas_call(
        _fwd_kernel,
        grid=(_NBLK,),
        in_specs=in_specs,
        out_specs=[
            pl.BlockSpec((1, 1), lambda i: (0, 0)),
            pl.BlockSpec((_D_IN, _BLK), lambda i: (0, i)),
        ],
        out_shape=[
            jax.ShapeDtypeStruct((1, 1), jnp.float32),
            jax.ShapeDtypeStruct((_D_IN, _B), jnp.float32),
        ],
        scratch_shapes=[
            pltpu.VMEM((_D_IN, _NCP), jnp.float32),
            pltpu.VMEM((5 * _D_LAT, _NCP), jnp.float32),
            pltpu.SMEM((4,), jnp.float32),
        ],
    )(xt, eW1, eb1, eW2, eb2, eW3t, eb3, dW1, db1, dW2, db2, dW3t, db3,
      qW1, qb1, qW2, qb2, qW3t, qb3, embt, nbmt)
    return loss, recon


def kernel(x, y, eW1, eb1, eW2, eb2, eW3, eb3, dW1, db1, dW2, db2, dW3, db3,
           qW1, qb1, qW2, qb2, qW3, qb3, embeddings):
    xt = x.reshape(_B, _D_IN).T
    embt = jnp.zeros((_D_LAT, _NCP), jnp.float32).at[:, :_NC].set(
        embeddings.reshape(_NC, _D_LAT).T)
    loss, recon_t = _forward_pallas(
        xt,
        eW1, eb1.reshape(-1, 1), eW2, eb2.reshape(-1, 1),
        eW3.T, eb3.reshape(-1, 1),
        dW1.astype(jnp.bfloat16), db1.reshape(-1, 1),
        dW2.astype(jnp.bfloat16), db2.reshape(-1, 1),
        dW3.astype(jnp.bfloat16).T, db3.reshape(-1, 1),
        qW1, qb1.reshape(-1, 1), qW2, qb2.reshape(-1, 1),
        qW3.T, qb3.reshape(-1, 1),
        embt, jnp.asarray(_NBMT))
    return loss[0, 0], recon_t.T.reshape(_B, 200, 8)


# speedup vs baseline: 1.0416x; 1.0416x over previous
import numpy as np
import jax
import jax.numpy as jnp
from jax.experimental import pallas as pl
from jax.experimental.pallas import tpu as pltpu

_B = 4096
_D_IN = 1600
_D_LAT = 64
_NC = 9
_NCP = 16
_BLK = 512
_NBLK = _B // _BLK

_C00 = (((0,), (0,)), ((), ()))
_STD = (((1,), (0,)), ((), ()))


def _neighbor_selection_mats_t():
    specs = [
        lambda k1, k2: (k1, k2),
        lambda k1, k2: (k1 + 1, k2) if k1 < 2 else None,
        lambda k1, k2: (k1 - 1, k2) if k1 > 0 else None,
        lambda k1, k2: (k1, k2 + 1) if k2 < 2 else None,
        lambda k1, k2: (k1, k2 - 1) if k2 > 0 else None,
    ]
    out = np.zeros((_NCP, 5 * _NCP), np.float32)
    for s, f in enumerate(specs):
        for j in range(_NC):
            r = f(j // 3, j % 3)
            if r is not None:
                out[r[0] * 3 + r[1], s * _NCP + j] = 1.0
    return out


_NBMT = _neighbor_selection_mats_t()


def _dot(a, b, dims):
    return jax.lax.dot_general(a, b, dims, preferred_element_type=jnp.float32)


def _fwd_kernel(xt_ref, eW1, eb1, eW2, eb2, eW3t, eb3,
                dW1, db1, dW2, db2, dW3t, db3,
                qW1, qb1, qW2, qb2, qW3t, qb3,
                embt_ref, nbmt_ref,
                loss_ref, recon_ref,
                xq_tab, nb_tab, acc):
    pid = pl.program_id(0)

    @pl.when(pid == 0)
    def _init():
        et = embt_ref[:]
        hq = jnp.maximum(_dot(qW1[:], et, _C00) + qb1[:], 0.0)
        hq = jnp.maximum(_dot(qW2[:], hq, _C00) + qb2[:], 0.0)
        xq_tab[:] = _dot(qW3t[:], hq, _STD) + qb3[:]
        nb_tab[:] = jnp.concatenate(
            [_dot(et, nbmt_ref[:, s * _NCP:(s + 1) * _NCP], _STD)
             for s in range(5)], axis=0)
        acc[0] = 0.0
        acc[1] = 0.0
        acc[2] = 0.0
        acc[3] = 0.0

    xt = xt_ref[:]
    h = jnp.maximum(_dot(eW1[:], xt, _C00) + eb1[:], 0.0)
    h = jnp.maximum(_dot(eW2[:], h, _C00) + eb2[:], 0.0)
    zt = jnp.tanh(_dot(eW3t[:], h, _STD) + eb3[:])

    et = embt_ref[:]
    best_d = jnp.sum((zt - et[:, 0:1]) ** 2, axis=0, keepdims=True)
    best_k = jnp.zeros((1, _BLK), jnp.int32)
    for j in range(1, _NC):
        dj = jnp.sum((zt - et[:, j:j + 1]) ** 2, axis=0, keepdims=True)
        upd = dj < best_d
        best_d = jnp.where(upd, dj, best_d)
        best_k = jnp.where(upd, j, best_k)

    onehot = (best_k == jax.lax.broadcasted_iota(
        jnp.int32, (_NCP, _BLK), 0)).astype(jnp.float32)

    xq = _dot(xq_tab[:], onehot, _STD)
    nb = _dot(nb_tab[:], onehot, _STD)
    zq = nb[0:_D_LAT, :]

    hd = jnp.maximum(_dot(dW1[:], zt, _C00) + db1[:], 0.0)
    hd = jnp.maximum(_dot(dW2[:], hd, _C00) + db2[:], 0.0)
    xe = _dot(dW3t[:], hd, _STD) + db3[:]

    recon_ref[:] = xq + xe

    z5 = jnp.concatenate([zt] * 5, axis=0)
    acc[0] += jnp.sum((xt - xe) ** 2)
    acc[1] += jnp.sum((xt - xq) ** 2)
    acc[2] += jnp.sum((zt - zq) ** 2)
    acc[3] += jnp.sum((z5 - nb) ** 2)

    @pl.when(pid == _NBLK - 1)
    def _fin():
        loss_val = ((acc[0] + acc[1]) / (_B * _D_IN)
                    + acc[2] / (_B * _D_LAT)
                    + acc[3] / (_B * 5 * _D_LAT))
        loss_ref[:, :] = jnp.full((1, 1), loss_val, jnp.float32)


def _const_spec(shape):
    return pl.BlockSpec(shape, lambda i: (0,) * len(shape))


def _forward_pallas(xt, eW1, eb1, eW2, eb2, eW3t, eb3,
                    dW1, db1, dW2, db2, dW3t, db3,
                    qW1, qb1, qW2, qb2, qW3t, qb3, embt, nbmt):
    in_specs = [pl.BlockSpec((_D_IN, _BLK), lambda i: (0, i))]
    for a in (eW1, eb1, eW2, eb2, eW3t, eb3, dW1, db1, dW2, db2, dW3t, db3,
              qW1, qb1, qW2, qb2, qW3t, qb3, embt, nbmt):
        in_specs.append(_const_spec(a.shape))
    loss, recon = pl.pallas_call(
        _fwd_kernel,
        grid=(_NBLK,),
        in_specs=in_specs,
        out_specs=[
            pl.BlockSpec((1, 1), lambda i: (0, 0)),
            pl.BlockSpec((_D_IN, _BLK), lambda i: (0, i)),
        ],
        out_shape=[
            jax.ShapeDtypeStruct((1, 1), jnp.float32),
            jax.ShapeDtypeStruct((_D_IN, _B), jnp.float32),
        ],
        scratch_shapes=[
            pltpu.VMEM((_D_IN, _NCP), jnp.float32),
            pltpu.VMEM((5 * _D_LAT, _NCP), jnp.float32),
            pltpu.SMEM((4,), jnp.float32),
        ],
    )(xt, eW1, eb1, eW2, eb2, eW3t, eb3, dW1, db1, dW2, db2, dW3t, db3,
      qW1, qb1, qW2, qb2, qW3t, qb3, embt, nbmt)
    return loss, recon


def kernel(x, y, eW1, eb1, eW2, eb2, eW3, eb3, dW1, db1, dW2, db2, dW3, db3,
           qW1, qb1, qW2, qb2, qW3, qb3, embeddings):
    xt = x.reshape(_B, _D_IN).T
    embt = jnp.zeros((_D_LAT, _NCP), jnp.float32).at[:, :_NC].set(
        embeddings.reshape(_NC, _D_LAT).T)
    loss, recon_t = _forward_pallas(
        xt,
        eW1, eb1.reshape(-1, 1), eW2, eb2.reshape(-1, 1),
        eW3.T, eb3.reshape(-1, 1),
        dW1, db1.reshape(-1, 1), dW2, db2.reshape(-1, 1),
        dW3.T, db3.reshape(-1, 1),
        qW1, qb1.reshape(-1, 1), qW2, qb2.reshape(-1, 1),
        qW3.T, qb3.reshape(-1, 1),
        embt, jnp.asarray(_NBMT))
    return loss[0, 0], recon_t.T.reshape(_B, 200, 8)


# speedup vs baseline: 1.1794x; 1.1323x over previous
import numpy as np
import jax
import jax.numpy as jnp
from jax.experimental import pallas as pl
from jax.experimental.pallas import tpu as pltpu

_B = 4096
_D_IN = 1600
_D_LAT = 64
_NC = 9
_NCP = 16
_BLK = 512
_NBLK = _B // _BLK

_C00 = (((0,), (0,)), ((), ()))
_STD = (((1,), (0,)), ((), ()))


def _neighbor_selection_mats_t():
    specs = [
        lambda k1, k2: (k1, k2),
        lambda k1, k2: (k1 + 1, k2) if k1 < 2 else None,
        lambda k1, k2: (k1 - 1, k2) if k1 > 0 else None,
        lambda k1, k2: (k1, k2 + 1) if k2 < 2 else None,
        lambda k1, k2: (k1, k2 - 1) if k2 > 0 else None,
    ]
    out = np.zeros((_NCP, 5 * _NCP), np.float32)
    for s, f in enumerate(specs):
        for j in range(_NC):
            r = f(j // 3, j % 3)
            if r is not None:
                out[r[0] * 3 + r[1], s * _NCP + j] = 1.0
    return out


_NBMT = _neighbor_selection_mats_t()


def _dot(a, b, dims):
    return jax.lax.dot_general(a, b, dims, preferred_element_type=jnp.float32)


def _fwd_kernel(xt_ref, eW1, eb1, eW2, eb2, eW3t, eb3,
                dW1, db1, dW2, db2, dW3t, db3,
                qW1, qb1, qW2, qb2, qW3t, qb3,
                embt_ref, nbmt_ref,
                loss_ref, recon_ref,
                xq_tab, nb_tab, bias_s, acc):
    pid = pl.program_id(0)

    @pl.when(pid == 0)
    def _init():
        for col, (row, n) in enumerate((
                (eb1, 1024), (eb2, 1024), (eb3, _D_LAT),
                (db1, 1024), (db2, 1024), (db3, _D_IN),
                (qb1, 1024), (qb2, 1024), (qb3, _D_IN))):
            bias_s[0:n, col:col + 1] = row[:].T

        et = embt_ref[:]
        hq = jnp.maximum(
            _dot(qW1[:], et, _C00) + bias_s[0:1024, 6:7], 0.0)
        hq = jnp.maximum(
            _dot(qW2[:], hq, _C00) + bias_s[0:1024, 7:8], 0.0)
        xq_tab[:] = _dot(qW3t[:], hq, _STD) + bias_s[0:_D_IN, 8:9]
        nb_tab[:] = jnp.concatenate(
            [_dot(et, nbmt_ref[:, s * _NCP:(s + 1) * _NCP], _STD)
             for s in range(5)], axis=0)
        acc[0] = 0.0
        acc[1] = 0.0
        acc[2] = 0.0
        acc[3] = 0.0

    xt = xt_ref[:]
    h = jnp.maximum(
        _dot(eW1[:], xt, _C00) + bias_s[0:1024, 0:1], 0.0)
    h = jnp.maximum(_dot(eW2[:], h, _C00) + bias_s[0:1024, 1:2], 0.0)
    zt = jnp.tanh(_dot(eW3t[:], h, _STD) + bias_s[0:_D_LAT, 2:3])

    et = embt_ref[:]
    best_d = jnp.sum((zt - et[:, 0:1]) ** 2, axis=0, keepdims=True)
    best_k = jnp.zeros((1, _BLK), jnp.int32)
    for j in range(1, _NC):
        dj = jnp.sum((zt - et[:, j:j + 1]) ** 2, axis=0, keepdims=True)
        upd = dj < best_d
        best_d = jnp.where(upd, dj, best_d)
        best_k = jnp.where(upd, j, best_k)

    onehot = (best_k == jax.lax.broadcasted_iota(
        jnp.int32, (_NCP, _BLK), 0)).astype(jnp.float32)

    xq = _dot(xq_tab[:], onehot, _STD)
    nb = _dot(nb_tab[:], onehot, _STD)
    zq = nb[0:_D_LAT, :]

    hd = jnp.maximum(_dot(dW1[:], zt, _C00) + bias_s[0:1024, 3:4], 0.0)
    hd = jnp.maximum(_dot(dW2[:], hd, _C00) + bias_s[0:1024, 4:5], 0.0)
    xe = _dot(dW3t[:], hd, _STD) + bias_s[0:_D_IN, 5:6]

    recon_ref[:] = xq + xe

    z5 = jnp.concatenate([zt] * 5, axis=0)
    acc[0] += jnp.sum((xt - xe) ** 2)
    acc[1] += jnp.sum((xt - xq) ** 2)
    acc[2] += jnp.sum((zt - zq) ** 2)
    acc[3] += jnp.sum((z5 - nb) ** 2)

    @pl.when(pid == _NBLK - 1)
    def _fin():
        loss_val = ((acc[0] + acc[1]) / (_B * _D_IN)
                    + acc[2] / (_B * _D_LAT)
                    + acc[3] / (_B * 5 * _D_LAT))
        loss_ref[:, :] = jnp.full((1, 1), loss_val, jnp.float32)


def _const_spec(shape):
    return pl.BlockSpec(shape, lambda i: (0,) * len(shape))


def _forward_pallas(xt, eW1, eb1, eW2, eb2, eW3t, eb3,
                    dW1, db1, dW2, db2, dW3t, db3,
                    qW1, qb1, qW2, qb2, qW3t, qb3, embt, nbmt):
    in_specs = [pl.BlockSpec((_D_IN, _BLK), lambda i: (0, i))]
    for a in (eW1, eb1, eW2, eb2, eW3t, eb3, dW1, db1, dW2, db2, dW3t, db3,
              qW1, qb1, qW2, qb2, qW3t, qb3, embt, nbmt):
        in_specs.append(_const_spec(a.shape))
    loss, recon = pl.pallas_call(
        _fwd_kernel,
        grid=(_NBLK,),
        in_specs=in_specs,
        out_specs=[
            pl.BlockSpec((1, 1), lambda i: (0, 0)),
            pl.BlockSpec((_D_IN, _BLK), lambda i: (0, i)),
        ],
        out_shape=[
            jax.ShapeDtypeStruct((1, 1), jnp.float32),
            jax.ShapeDtypeStruct((_D_IN, _B), jnp.float32),
        ],
        scratch_shapes=[
            pltpu.VMEM((_D_IN, _NCP), jnp.float32),
            pltpu.VMEM((5 * _D_LAT, _NCP), jnp.float32),
            pltpu.VMEM((_D_IN, _NCP), jnp.float32),
            pltpu.SMEM((4,), jnp.float32),
        ],
    )(xt, eW1, eb1, eW2, eb2, eW3t, eb3, dW1, db1, dW2, db2, dW3t, db3,
      qW1, qb1, qW2, qb2, qW3t, qb3, embt, nbmt)
    return loss, recon


def kernel(x, y, eW1, eb1, eW2, eb2, eW3, eb3, dW1, db1, dW2, db2, dW3, db3,
           qW1, qb1, qW2, qb2, qW3, qb3, embeddings):
    xt = x.reshape(_B, _D_IN).T
    embt = jnp.zeros((_D_LAT, _NCP), jnp.float32).at[:, :_NC].set(
        embeddings.reshape(_NC, _D_LAT).T)
    loss, recon_t = _forward_pallas(
        xt,
        eW1, eb1.reshape(1, -1), eW2, eb2.reshape(1, -1),
        eW3.T, eb3.reshape(1, -1),
        dW1, db1.reshape(1, -1), dW2, db2.reshape(1, -1),
        dW3.T, db3.reshape(1, -1),
        qW1, qb1.reshape(1, -1), qW2, qb2.reshape(1, -1),
        qW3.T, qb3.reshape(1, -1),
        embt, jnp.asarray(_NBMT))
    return loss[0, 0], recon_t.T.reshape(_B, 200, 8)


# speedup vs baseline: 1.2002x; 1.0176x over previous
import numpy as np
import jax
import jax.numpy as jnp
from jax.experimental import pallas as pl
from jax.experimental.pallas import tpu as pltpu

_B = 4096
_D_IN = 1600
_D_LAT = 64
_NC = 9
_NCP = 16
_BLK = 512
_NBLK = _B // _BLK

_C00 = (((0,), (0,)), ((), ()))
_STD = (((1,), (0,)), ((), ()))


def _neighbor_selection_mats_t():
    specs = [
        lambda k1, k2: (k1, k2),
        lambda k1, k2: (k1 + 1, k2) if k1 < 2 else None,
        lambda k1, k2: (k1 - 1, k2) if k1 > 0 else None,
        lambda k1, k2: (k1, k2 + 1) if k2 < 2 else None,
        lambda k1, k2: (k1, k2 - 1) if k2 > 0 else None,
    ]
    out = np.zeros((_NCP, 5 * _NCP), np.float32)
    for s, f in enumerate(specs):
        for j in range(_NC):
            r = f(j // 3, j % 3)
            if r is not None:
                out[r[0] * 3 + r[1], s * _NCP + j] = 1.0
    return out


_NBMT = _neighbor_selection_mats_t()


def _dot(a, b, dims):
    return jax.lax.dot_general(a, b, dims, preferred_element_type=jnp.float32)


def _fwd_kernel(xt_ref, eW1, eb1, eW2, eb2, eW3t, eb3,
                dW1, db1, dW2, db2, dW3t, db3,
                qW1, qb1, qW2, qb2, qW3t, qb3,
                emb9_ref, nbmt_ref,
                loss_ref, recon_ref,
                xq_tab, nb_tab, bias_s, et_s, acc):
    pid = pl.program_id(0)

    @pl.when(pid == 0)
    def _init():
        for col, (row, n) in enumerate((
                (eb1, 1024), (eb2, 1024), (eb3, _D_LAT),
                (db1, 1024), (db2, 1024), (db3, _D_IN),
                (qb1, 1024), (qb2, 1024), (qb3, _D_IN))):
            bias_s[0:n, col:col + 1] = row[:].T

        et = jnp.concatenate(
            [emb9_ref[:].T,
             jnp.zeros((_D_LAT, _NCP - _NC), jnp.float32)], axis=1)
        et_s[:] = et
        hq = jnp.maximum(
            _dot(qW1[:], et, _C00) + bias_s[0:1024, 6:7], 0.0)
        hq = jnp.maximum(
            _dot(qW2[:], hq, _C00) + bias_s[0:1024, 7:8], 0.0)
        xq_tab[:] = _dot(qW3t[:], hq, _STD) + bias_s[0:_D_IN, 8:9]
        nb_tab[:] = jnp.concatenate(
            [_dot(et, nbmt_ref[:, s * _NCP:(s + 1) * _NCP], _STD)
             for s in range(5)], axis=0)
        acc[0] = 0.0
        acc[1] = 0.0
        acc[2] = 0.0
        acc[3] = 0.0

    xt = xt_ref[:]
    h = jnp.maximum(
        _dot(eW1[:], xt, _C00) + bias_s[0:1024, 0:1], 0.0)
    h = jnp.maximum(_dot(eW2[:], h, _C00) + bias_s[0:1024, 1:2], 0.0)
    zt = jnp.tanh(_dot(eW3t[:], h, _STD) + bias_s[0:_D_LAT, 2:3])

    et = et_s[:]
    best_d = jnp.sum((zt - et[:, 0:1]) ** 2, axis=0, keepdims=True)
    best_k = jnp.zeros((1, _BLK), jnp.int32)
    for j in range(1, _NC):
        dj = jnp.sum((zt - et[:, j:j + 1]) ** 2, axis=0, keepdims=True)
        upd = dj < best_d
        best_d = jnp.where(upd, dj, best_d)
        best_k = jnp.where(upd, j, best_k)

    onehot = (best_k == jax.lax.broadcasted_iota(
        jnp.int32, (_NCP, _BLK), 0)).astype(jnp.float32)

    xq = _dot(xq_tab[:], onehot, _STD)
    nb = _dot(nb_tab[:], onehot, _STD)
    zq = nb[0:_D_LAT, :]

    hd = jnp.maximum(_dot(dW1[:], zt, _C00) + bias_s[0:1024, 3:4], 0.0)
    hd = jnp.maximum(_dot(dW2[:], hd, _C00) + bias_s[0:1024, 4:5], 0.0)
    xe = _dot(dW3t[:], hd, _STD) + bias_s[0:_D_IN, 5:6]

    recon_ref[:] = xq + xe

    z5 = jnp.concatenate([zt] * 5, axis=0)
    acc[0] += jnp.sum((xt - xe) ** 2)
    acc[1] += jnp.sum((xt - xq) ** 2)
    acc[2] += jnp.sum((zt - zq) ** 2)
    acc[3] += jnp.sum((z5 - nb) ** 2)

    @pl.when(pid == _NBLK - 1)
    def _fin():
        loss_val = ((acc[0] + acc[1]) / (_B * _D_IN)
                    + acc[2] / (_B * _D_LAT)
                    + acc[3] / (_B * 5 * _D_LAT))
        loss_ref[:, :] = jnp.full((1, 1), loss_val, jnp.float32)


def _const_spec(shape):
    return pl.BlockSpec(shape, lambda i: (0,) * len(shape))


def _forward_pallas(xt, eW1, eb1, eW2, eb2, eW3t, eb3,
                    dW1, db1, dW2, db2, dW3t, db3,
                    qW1, qb1, qW2, qb2, qW3t, qb3, embt, nbmt):
    in_specs = [pl.BlockSpec((_D_IN, _BLK), lambda i: (0, i))]
    for a in (eW1, eb1, eW2, eb2, eW3t, eb3, dW1, db1, dW2, db2, dW3t, db3,
              qW1, qb1, qW2, qb2, qW3t, qb3, embt, nbmt):
        in_specs.append(_const_spec(a.shape))
    loss, recon = pl.pallas_call(
        _fwd_kernel,
        grid=(_NBLK,),
        in_specs=in_specs,
        out_specs=[
            pl.BlockSpec((1, 1), lambda i: (0, 0)),
            pl.BlockSpec((_D_IN, _BLK), lambda i: (0, i)),
        ],
        out_shape=[
            jax.ShapeDtypeStruct((1, 1), jnp.float32),
            jax.ShapeDtypeStruct((_D_IN, _B), jnp.float32),
        ],
        scratch_shapes=[
            pltpu.VMEM((_D_IN, _NCP), jnp.float32),
            pltpu.VMEM((5 * _D_LAT, _NCP), jnp.float32),
            pltpu.VMEM((_D_IN, _NCP), jnp.float32),
            pltpu.VMEM((_D_LAT, _NCP), jnp.float32),
            pltpu.SMEM((4,), jnp.float32),
        ],
    )(xt, eW1, eb1, eW2, eb2, eW3t, eb3, dW1, db1, dW2, db2, dW3t, db3,
      qW1, qb1, qW2, qb2, qW3t, qb3, embt, nbmt)
    return loss, recon


def kernel(x, y, eW1, eb1, eW2, eb2, eW3, eb3, dW1, db1, dW2, db2, dW3, db3,
           qW1, qb1, qW2, qb2, qW3, qb3, embeddings):
    xt = x.reshape(_B, _D_IN).T
    embt = embeddings.reshape(_NC, _D_LAT)
    loss, recon_t = _forward_pallas(
        xt,
        eW1, eb1.reshape(1, -1), eW2, eb2.reshape(1, -1),
        eW3.T, eb3.reshape(1, -1),
        dW1, db1.reshape(1, -1), dW2, db2.reshape(1, -1),
        dW3.T, db3.reshape(1, -1),
        qW1, qb1.reshape(1, -1), qW2, qb2.reshape(1, -1),
        qW3.T, qb3.reshape(1, -1),
        embt, jnp.asarray(_NBMT))
    return loss[0, 0], recon_t.T.reshape(_B, 200, 8)
